# baseline (speedup 1.0000x reference)
import functools

import jax
import jax.numpy as jnp
from jax import lax
from jax.experimental import pallas as pl
from jax.experimental.pallas import tpu as pltpu
from jax.experimental.pallas import tpu_sc as plsc

N_DATASETS = 3
NUM_USERS = 100000
DIM = 64
BATCH = 16384

NUM_CORES = 2
NUM_SUBCORES = 16
NUM_WORKERS = NUM_CORES * NUM_SUBCORES
B_PER_W = BATCH // NUM_WORKERS
ROWS_PER_W = B_PER_W * N_DATASETS
CHUNK = 128
N_CHUNKS = ROWS_PER_W // CHUNK
LANES = 16
N_GROUPS = ROWS_PER_W // LANES


def _sc_gather(table_flat, idx_flat):
  mesh = plsc.VectorSubcoreMesh(core_axis_name="c", subcore_axis_name="s")

  @functools.partial(
      pl.kernel,
      out_type=jax.ShapeDtypeStruct((BATCH * N_DATASETS, DIM), jnp.float32),
      mesh=mesh,
      scratch_types=[
          pltpu.VMEM((B_PER_W,), jnp.int32),
          pltpu.VMEM((N_CHUNKS, CHUNK), jnp.int32),
          pltpu.VMEM((ROWS_PER_W, DIM), jnp.float32),
          pltpu.SemaphoreType.DMA,
      ],
      compiler_params=pltpu.CompilerParams(
          use_tc_tiling_on_sc=False, needs_layout_passes=False),
  )
  def k(tab_hbm, idx_hbm, out_hbm, idx_v, idx3_v, rows_v, sem):
    wid = lax.axis_index("s") * NUM_CORES + lax.axis_index("c")
    pltpu.sync_copy(idx_hbm.at[pl.ds(wid * B_PER_W, B_PER_W)], idx_v)

    io = lax.iota(jnp.int32, LANES)
    for h in range(B_PER_W // LANES):
      v = idx_v[pl.ds(h * LANES, LANES)]
      i3 = (io + (h * LANES)) * 3
      for d in range(N_DATASETS):
        r = i3 + d
        plsc.store_scatter(idx3_v, [r >> 7, r & 127], v + d * NUM_USERS)

    copies = []
    for c in range(N_CHUNKS):
      cp = pltpu.make_async_copy(
          tab_hbm.at[idx3_v.at[c]],
          rows_v.at[pl.ds(c * CHUNK, CHUNK)],
          sem,
      )
      cp.start()
      copies.append(cp)
    for cp in copies:
      cp.wait()

    pltpu.sync_copy(rows_v, out_hbm.at[pl.ds(wid * ROWS_PER_W, ROWS_PER_W)])

  return k(table_flat, idx_flat)


def kernel(user_embeds_list, userIdx):
  table_flat = user_embeds_list.reshape(N_DATASETS * NUM_USERS, DIM)
  idx_flat = userIdx.astype(jnp.int32)
  out = _sc_gather(table_flat, idx_flat)
  return out.reshape(BATCH, N_DATASETS * DIM)


# speedup vs baseline: 2.0637x; 2.0637x over previous
import functools

import jax
import jax.numpy as jnp
from jax import lax
from jax.experimental import pallas as pl
from jax.experimental.pallas import tpu as pltpu
from jax.experimental.pallas import tpu_sc as plsc

N_DATASETS = 3
NUM_USERS = 100000
DIM = 64
BATCH = 16384

NUM_CORES = 2
NUM_SUBCORES = 16
NUM_WORKERS = NUM_CORES * NUM_SUBCORES
N_COLS = N_DATASETS * DIM
COLS_PER_W = N_COLS // NUM_WORKERS
LANES = 16
HALF = BATCH // 2


def _sc_gather(table_t, idx_flat):
  mesh = plsc.VectorSubcoreMesh(core_axis_name="c", subcore_axis_name="s")

  @functools.partial(
      pl.kernel,
      out_type=jax.ShapeDtypeStruct((N_COLS, BATCH), jnp.float32),
      mesh=mesh,
      scratch_types=[
          pltpu.VMEM((BATCH,), jnp.int32),
          pltpu.VMEM((NUM_USERS,), jnp.float32),
          pltpu.VMEM((HALF,), jnp.float32),
      ],
      compiler_params=pltpu.CompilerParams(
          use_tc_tiling_on_sc=True, needs_layout_passes=False),
  )
  def k(tab_hbm, idx_hbm, out_hbm, idx_v, row_v, out_v):
    wid = lax.axis_index("s") * NUM_CORES + lax.axis_index("c")
    pltpu.sync_copy(idx_hbm, idx_v)

    for j in range(COLS_PER_W):
      col = wid * COLS_PER_W + j
      d = col // DIM
      f = col - d * DIM
      pltpu.sync_copy(tab_hbm.at[d, f], row_v)

      for half in range(2):
        def body(v, _):
          u16 = idx_v[pl.ds(half * HALF + v * LANES, LANES)]
          out_v[pl.ds(v * LANES, LANES)] = plsc.load_gather(row_v, [u16])
          return _
        lax.fori_loop(0, HALF // LANES, body, 0, unroll=4)
        pltpu.sync_copy(out_v, out_hbm.at[col, pl.ds(half * HALF, HALF)])

  return k(table_t, idx_flat)


def kernel(user_embeds_list, userIdx):
  table_t = jnp.transpose(user_embeds_list, (0, 2, 1))
  idx_flat = userIdx.astype(jnp.int32)
  out_t = _sc_gather(table_t, idx_flat)
  return jnp.transpose(out_t)


# speedup vs baseline: 3.5072x; 1.6995x over previous
import functools

import jax
import jax.numpy as jnp
from jax import lax
from jax.experimental import pallas as pl
from jax.experimental.pallas import tpu as pltpu
from jax.experimental.pallas import tpu_sc as plsc

N_DATASETS = 3
NUM_USERS = 100000
DIM = 64
BATCH = 16384

NUM_CORES = 2
NUM_SUBCORES = 16
NUM_WORKERS = NUM_CORES * NUM_SUBCORES
N_COLS = N_DATASETS * DIM
COLS_PER_W = N_COLS // NUM_WORKERS
LANES = 16
HALF = BATCH // 2


def _sc_gather(table_t, idx_flat):
  mesh = plsc.VectorSubcoreMesh(core_axis_name="c", subcore_axis_name="s")

  @functools.partial(
      pl.kernel,
      out_type=jax.ShapeDtypeStruct((N_COLS, BATCH), jnp.float32),
      mesh=mesh,
      scratch_types=[
          pltpu.VMEM((BATCH,), jnp.int32),
          pltpu.VMEM((NUM_USERS,), jnp.float32),
          pltpu.VMEM((HALF,), jnp.float32),
      ],
      compiler_params=pltpu.CompilerParams(
          use_tc_tiling_on_sc=True, needs_layout_passes=False),
  )
  def k(tab_hbm, idx_hbm, out_hbm, idx_v, row_v, out_v):
    wid = lax.axis_index("s") * NUM_CORES + lax.axis_index("c")
    pltpu.sync_copy(idx_hbm, idx_v)

    for j in range(COLS_PER_W):
      col = wid * COLS_PER_W + j
      d = col // DIM
      f = col - d * DIM
      pltpu.sync_copy(tab_hbm.at[d, f], row_v)

      for half in range(2):
        @plsc.parallel_loop(0, HALF // LANES, unroll=8)
        def body(v):
          u16 = idx_v[pl.ds(half * HALF + v * LANES, LANES)]
          out_v[pl.ds(v * LANES, LANES)] = plsc.load_gather(row_v, [u16])
        pltpu.sync_copy(out_v, out_hbm.at[col, pl.ds(half * HALF, HALF)])

  return k(table_t, idx_flat)


def kernel(user_embeds_list, userIdx):
  table_t = jnp.transpose(user_embeds_list, (0, 2, 1))
  idx_flat = userIdx.astype(jnp.int32)
  out_t = _sc_gather(table_t, idx_flat)
  return jnp.transpose(out_t)
